# baseline (device time: 61481 ns/iter reference)
import jax
import jax.numpy as jnp
from jax import lax
from jax.experimental import pallas as pl
from jax.experimental.pallas import tpu as pltpu

W = 8
M_PER = 512
D = 512
EPS = 1e-6


def kernel(partial, gamma):
    m_tot = partial.shape[1]
    assert partial.shape == (1, W * M_PER, D), partial.shape
    x = partial.reshape(m_tot, D)
    gamma2d = gamma.reshape(1, D)

    def body(x_ref, g_ref, out_ref, send_buf, recv_buf, send_sems, recv_sems):
        my = lax.axis_index("i")
        left = lax.rem(my + W - 1, W)
        right = lax.rem(my + 1, W)

        barrier_sem = pltpu.get_barrier_semaphore()
        for nbr in (left, right):
            pl.semaphore_signal(
                barrier_sem, inc=1,
                device_id=(nbr,), device_id_type=pl.DeviceIdType.MESH,
            )
        pl.semaphore_wait(barrier_sem, 2)

        c0 = lax.rem(my + W - 1, W)
        send_buf[0] = x_ref[pl.ds(c0 * M_PER, M_PER), :].astype(jnp.bfloat16)

        for s in range(W - 1):
            rdma = pltpu.make_async_remote_copy(
                src_ref=send_buf.at[s],
                dst_ref=recv_buf.at[s],
                send_sem=send_sems.at[s],
                recv_sem=recv_sems.at[s],
                device_id=(right,),
                device_id_type=pl.DeviceIdType.MESH,
            )
            rdma.start()
            rdma.wait()

            c = lax.rem(my + 2 * W - s - 2, W)
            acc = (recv_buf[s].astype(jnp.float32)
                   + x_ref[pl.ds(c * M_PER, M_PER), :])
            if s < W - 2:
                send_buf[s + 1] = acc.astype(jnp.bfloat16)
            else:
                rms = jnp.sqrt(
                    jnp.mean(acc * acc, axis=-1, keepdims=True) + EPS)
                out_ref[...] = acc / rms * g_ref[...]

    return pl.pallas_call(
        body,
        out_shape=jax.ShapeDtypeStruct((M_PER, D), jnp.float32),
        in_specs=[
            pl.BlockSpec(memory_space=pltpu.VMEM),
            pl.BlockSpec(memory_space=pltpu.VMEM),
        ],
        out_specs=pl.BlockSpec(memory_space=pltpu.VMEM),
        scratch_shapes=[
            pltpu.VMEM((W - 1, M_PER, D), jnp.bfloat16),
            pltpu.VMEM((W - 1, M_PER, D), jnp.bfloat16),
            pltpu.SemaphoreType.DMA((W - 1,)),
            pltpu.SemaphoreType.DMA((W - 1,)),
        ],
        compiler_params=pltpu.CompilerParams(collective_id=0),
    )(x, gamma2d)


# device time: 44749 ns/iter; 1.3739x vs baseline; 1.3739x over previous
import jax
import jax.numpy as jnp
from jax import lax
from jax.experimental import pallas as pl
from jax.experimental.pallas import tpu as pltpu

W = 8
M_PER = 512
H = M_PER // 2
D = 512
EPS = 1e-6


def kernel(partial, gamma):
    m_tot = partial.shape[1]
    assert partial.shape == (1, W * M_PER, D), partial.shape
    x = partial.reshape(m_tot, D)
    gamma2d = gamma.reshape(1, D)

    def body(x_ref, g_ref, out_ref,
             send_cw, recv_cw, send_ccw, recv_ccw,
             ssem_cw, rsem_cw, ssem_ccw, rsem_ccw):
        my = lax.axis_index("i")
        left = lax.rem(my + W - 1, W)
        right = lax.rem(my + 1, W)

        barrier_sem = pltpu.get_barrier_semaphore()
        for nbr in (left, right):
            pl.semaphore_signal(
                barrier_sem, inc=1,
                device_id=(nbr,), device_id_type=pl.DeviceIdType.MESH,
            )
        pl.semaphore_wait(barrier_sem, 2)

        c_cw0 = lax.rem(my + W - 1, W)
        c_ccw0 = lax.rem(my + 1, W)
        send_cw[0] = x_ref[pl.ds(c_cw0 * M_PER, H), :].astype(jnp.bfloat16)
        send_ccw[0] = x_ref[pl.ds(c_ccw0 * M_PER + H, H), :].astype(jnp.bfloat16)

        for s in range(W - 1):
            rdma_cw = pltpu.make_async_remote_copy(
                src_ref=send_cw.at[s], dst_ref=recv_cw.at[s],
                send_sem=ssem_cw.at[s], recv_sem=rsem_cw.at[s],
                device_id=(right,), device_id_type=pl.DeviceIdType.MESH,
            )
            rdma_ccw = pltpu.make_async_remote_copy(
                src_ref=send_ccw.at[s], dst_ref=recv_ccw.at[s],
                send_sem=ssem_ccw.at[s], recv_sem=rsem_ccw.at[s],
                device_id=(left,), device_id_type=pl.DeviceIdType.MESH,
            )
            rdma_cw.start()
            rdma_ccw.start()

            c_cw = lax.rem(my + 2 * W - s - 2, W)
            c_ccw = lax.rem(my + s + 2, W)

            rdma_cw.wait()
            acc_cw = (recv_cw[s].astype(jnp.float32)
                      + x_ref[pl.ds(c_cw * M_PER, H), :])
            if s < W - 2:
                send_cw[s + 1] = acc_cw.astype(jnp.bfloat16)

            rdma_ccw.wait()
            acc_ccw = (recv_ccw[s].astype(jnp.float32)
                       + x_ref[pl.ds(c_ccw * M_PER + H, H), :])
            if s < W - 2:
                send_ccw[s + 1] = acc_ccw.astype(jnp.bfloat16)

            if s == W - 2:
                rms_t = jnp.sqrt(
                    jnp.mean(acc_cw * acc_cw, axis=-1, keepdims=True) + EPS)
                out_ref[0:H, :] = acc_cw / rms_t * g_ref[...]
                rms_b = jnp.sqrt(
                    jnp.mean(acc_ccw * acc_ccw, axis=-1, keepdims=True) + EPS)
                out_ref[H:M_PER, :] = acc_ccw / rms_b * g_ref[...]

    return pl.pallas_call(
        body,
        out_shape=jax.ShapeDtypeStruct((M_PER, D), jnp.float32),
        in_specs=[
            pl.BlockSpec(memory_space=pltpu.VMEM),
            pl.BlockSpec(memory_space=pltpu.VMEM),
        ],
        out_specs=pl.BlockSpec(memory_space=pltpu.VMEM),
        scratch_shapes=[
            pltpu.VMEM((W - 1, H, D), jnp.bfloat16),
            pltpu.VMEM((W - 1, H, D), jnp.bfloat16),
            pltpu.VMEM((W - 1, H, D), jnp.bfloat16),
            pltpu.VMEM((W - 1, H, D), jnp.bfloat16),
            pltpu.SemaphoreType.DMA((W - 1,)),
            pltpu.SemaphoreType.DMA((W - 1,)),
            pltpu.SemaphoreType.DMA((W - 1,)),
            pltpu.SemaphoreType.DMA((W - 1,)),
        ],
        compiler_params=pltpu.CompilerParams(collective_id=0),
    )(x, gamma2d)


# device time: 28538 ns/iter; 2.1544x vs baseline; 1.5680x over previous
import jax
import jax.numpy as jnp
from jax import lax
from jax.experimental import pallas as pl
from jax.experimental.pallas import tpu as pltpu

W = 8
M_PER = 512
D = 512
EPS = 1e-6

ORDERINGS = (
    ((4, 3, 1), 0, 176),
    ((1, 4, 3), 176, 176),
    ((3, 1, 4), 352, 160),
)
NEIGHBOR_MASKS = (1, 3, 4)


def kernel(partial, gamma):
    m_tot = partial.shape[1]
    assert partial.shape == (1, W * M_PER, D), partial.shape
    x = partial.reshape(m_tot, D)
    gamma2d = gamma.reshape(1, D)

    def body(x_ref, g_ref, out_ref, *scratch):
        bufs = [scratch[5 * k: 5 * k + 5] for k in range(3)]
        ssem, rsem = scratch[15], scratch[16]

        my = lax.axis_index("i")

        barrier_sem = pltpu.get_barrier_semaphore()
        for m in NEIGHBOR_MASKS:
            pl.semaphore_signal(
                barrier_sem, inc=1,
                device_id=(my ^ m,), device_id_type=pl.DeviceIdType.MESH,
            )
        pl.semaphore_wait(barrier_sem, len(NEIGHBOR_MASKS))

        def stripe(c, off, h):
            return x_ref[pl.ds(c * M_PER + off, h), :]

        rdmas1 = []
        for k, ((m1, m2, m3), off, h) in enumerate(ORDERINGS):
            send1, recv1, acc, recv2, recv3 = bufs[k]
            span = (0, m3, m2, m2 ^ m3)
            for j, s in enumerate(span):
                send1[j] = stripe(my ^ m1 ^ s, off, h).astype(jnp.bfloat16)
            r = pltpu.make_async_remote_copy(
                src_ref=send1, dst_ref=recv1,
                send_sem=ssem.at[k, 0], recv_sem=rsem.at[k, 0],
                device_id=(my ^ m1,), device_id_type=pl.DeviceIdType.MESH,
            )
            r.start()
            rdmas1.append(r)

        for k, ((m1, m2, m3), off, h) in enumerate(ORDERINGS):
            send1, recv1, acc, recv2, recv3 = bufs[k]
            span = (0, m3, m2, m2 ^ m3)
            rdmas1[k].wait()
            for j, s in enumerate(span):
                acc[j] = (recv1[j].astype(jnp.float32)
                          + stripe(my ^ s, off, h)).astype(jnp.bfloat16)

        rdmas2 = []
        for k, ((m1, m2, m3), off, h) in enumerate(ORDERINGS):
            send1, recv1, acc, recv2, recv3 = bufs[k]
            r = pltpu.make_async_remote_copy(
                src_ref=acc.at[pl.ds(2, 2)], dst_ref=recv2,
                send_sem=ssem.at[k, 1], recv_sem=rsem.at[k, 1],
                device_id=(my ^ m2,), device_id_type=pl.DeviceIdType.MESH,
            )
            r.start()
            rdmas2.append(r)
        for k, ((m1, m2, m3), off, h) in enumerate(ORDERINGS):
            send1, recv1, acc, recv2, recv3 = bufs[k]
            rdmas2[k].wait()
            for j in range(2):
                acc[j] = (acc[j].astype(jnp.float32)
                          + recv2[j].astype(jnp.float32)).astype(jnp.bfloat16)

        rdmas3 = []
        for k, ((m1, m2, m3), off, h) in enumerate(ORDERINGS):
            send1, recv1, acc, recv2, recv3 = bufs[k]
            r = pltpu.make_async_remote_copy(
                src_ref=acc.at[pl.ds(1, 1)], dst_ref=recv3,
                send_sem=ssem.at[k, 2], recv_sem=rsem.at[k, 2],
                device_id=(my ^ m3,), device_id_type=pl.DeviceIdType.MESH,
            )
            r.start()
            rdmas3.append(r)

        for k, ((m1, m2, m3), off, h) in enumerate(ORDERINGS):
            send1, recv1, acc, recv2, recv3 = bufs[k]
            rdmas3[k].wait()
            y = acc[0].astype(jnp.float32) + recv3[0].astype(jnp.float32)
            rms = jnp.sqrt(jnp.mean(y * y, axis=-1, keepdims=True) + EPS)
            out_ref[pl.ds(off, h), :] = y / rms * g_ref[...]

    scratch_shapes = []
    for (_, _, h) in ORDERINGS:
        scratch_shapes += [
            pltpu.VMEM((4, h, D), jnp.bfloat16),
            pltpu.VMEM((4, h, D), jnp.bfloat16),
            pltpu.VMEM((4, h, D), jnp.bfloat16),
            pltpu.VMEM((2, h, D), jnp.bfloat16),
            pltpu.VMEM((1, h, D), jnp.bfloat16),
        ]
    scratch_shapes += [
        pltpu.SemaphoreType.DMA((3, 3)),
        pltpu.SemaphoreType.DMA((3, 3)),
    ]

    return pl.pallas_call(
        body,
        out_shape=jax.ShapeDtypeStruct((M_PER, D), jnp.float32),
        in_specs=[
            pl.BlockSpec(memory_space=pltpu.VMEM),
            pl.BlockSpec(memory_space=pltpu.VMEM),
        ],
        out_specs=pl.BlockSpec(memory_space=pltpu.VMEM),
        scratch_shapes=scratch_shapes,
        compiler_params=pltpu.CompilerParams(collective_id=0),
    )(x, gamma2d)


# device time: 24677 ns/iter; 2.4914x vs baseline; 1.1565x over previous
import jax
import jax.numpy as jnp
from jax import lax
from jax.experimental import pallas as pl
from jax.experimental.pallas import tpu as pltpu

W = 8
M_PER = 512
D = 512
EPS = 1e-6

ORDERINGS = (
    ((4, 3, 1), 0, 176),
    ((1, 4, 3), 176, 176),
    ((3, 1, 4), 352, 160),
)
NEIGHBOR_MASKS = (1, 3, 4)
NMSG = 5


def kernel(partial, gamma):
    m_tot = partial.shape[1]
    assert partial.shape == (1, W * M_PER, D), partial.shape
    x = partial.reshape(m_tot, D)
    gamma2d = gamma.reshape(1, D)

    def body(x_ref, g_ref, out_ref, *scratch):
        bufs = [scratch[5 * k: 5 * k + 5] for k in range(3)]
        ssem, rsem = scratch[15], scratch[16]

        my = lax.axis_index("i")

        barrier_sem = pltpu.get_barrier_semaphore()
        for m in NEIGHBOR_MASKS:
            pl.semaphore_signal(
                barrier_sem, inc=1,
                device_id=(my ^ m,), device_id_type=pl.DeviceIdType.MESH,
            )
        pl.semaphore_wait(barrier_sem, len(NEIGHBOR_MASKS))

        def stripe(c, off, h):
            return x_ref[pl.ds(c * M_PER + off, h), :]

        def mk(k, msg, src, dst, partner):
            return pltpu.make_async_remote_copy(
                src_ref=src, dst_ref=dst,
                send_sem=ssem.at[k, msg], recv_sem=rsem.at[k, msg],
                device_id=(partner,), device_id_type=pl.DeviceIdType.MESH,
            )

        spans = [(0, m3, m2, m2 ^ m3) for ((m1, m2, m3), _, _) in ORDERINGS]
        r1a, r1b, r2a, r2b, r3 = ({} for _ in range(5))

        for k, ((m1, m2, m3), off, h) in enumerate(ORDERINGS):
            send1, recv1, acc, recv2, recv3 = bufs[k]
            for j in (2, 3):
                send1[j] = stripe(my ^ m1 ^ spans[k][j], off, h).astype(
                    jnp.bfloat16)
            r1a[k] = mk(k, 0, send1.at[pl.ds(2, 2)], recv1.at[pl.ds(2, 2)],
                        my ^ m1)
            r1a[k].start()
        for k, ((m1, m2, m3), off, h) in enumerate(ORDERINGS):
            send1, recv1, acc, recv2, recv3 = bufs[k]
            for j in (0, 1):
                send1[j] = stripe(my ^ m1 ^ spans[k][j], off, h).astype(
                    jnp.bfloat16)
            r1b[k] = mk(k, 1, send1.at[pl.ds(0, 2)], recv1.at[pl.ds(0, 2)],
                        my ^ m1)
            r1b[k].start()

        for k, ((m1, m2, m3), off, h) in enumerate(ORDERINGS):
            send1, recv1, acc, recv2, recv3 = bufs[k]
            r1a[k].wait()
            for j in (3, 2):
                acc[j] = (recv1[j].astype(jnp.float32)
                          + stripe(my ^ spans[k][j], off, h)
                          ).astype(jnp.bfloat16)
            r2a[k] = mk(k, 2, acc.at[pl.ds(3, 1)], recv2.at[pl.ds(1, 1)],
                        my ^ m2)
            r2b[k] = mk(k, 3, acc.at[pl.ds(2, 1)], recv2.at[pl.ds(0, 1)],
                        my ^ m2)
            r2a[k].start()
            r2b[k].start()

        for k, ((m1, m2, m3), off, h) in enumerate(ORDERINGS):
            send1, recv1, acc, recv2, recv3 = bufs[k]
            r1b[k].wait()
            for j in (1, 0):
                acc[j] = (recv1[j].astype(jnp.float32)
                          + stripe(my ^ spans[k][j], off, h)
                          ).astype(jnp.bfloat16)

        for k, ((m1, m2, m3), off, h) in enumerate(ORDERINGS):
            send1, recv1, acc, recv2, recv3 = bufs[k]
            r2a[k].wait()
            acc[1] = (acc[1].astype(jnp.float32)
                      + recv2[1].astype(jnp.float32)).astype(jnp.bfloat16)
            r3[k] = mk(k, 4, acc.at[pl.ds(1, 1)], recv3, my ^ m3)
            r3[k].start()

        for k, ((m1, m2, m3), off, h) in enumerate(ORDERINGS):
            send1, recv1, acc, recv2, recv3 = bufs[k]
            r2b[k].wait()
            acc[0] = (acc[0].astype(jnp.float32)
                      + recv2[0].astype(jnp.float32)).astype(jnp.bfloat16)

        for k, ((m1, m2, m3), off, h) in enumerate(ORDERINGS):
            send1, recv1, acc, recv2, recv3 = bufs[k]
            r3[k].wait()
            y = acc[0].astype(jnp.float32) + recv3[0].astype(jnp.float32)
            rms = jnp.sqrt(jnp.mean(y * y, axis=-1, keepdims=True) + EPS)
            out_ref[pl.ds(off, h), :] = y / rms * g_ref[...]

    scratch_shapes = []
    for (_, _, h) in ORDERINGS:
        scratch_shapes += [
            pltpu.VMEM((4, h, D), jnp.bfloat16),
            pltpu.VMEM((4, h, D), jnp.bfloat16),
            pltpu.VMEM((4, h, D), jnp.bfloat16),
            pltpu.VMEM((2, h, D), jnp.bfloat16),
            pltpu.VMEM((1, h, D), jnp.bfloat16),
        ]
    scratch_shapes += [
        pltpu.SemaphoreType.DMA((3, NMSG)),
        pltpu.SemaphoreType.DMA((3, NMSG)),
    ]

    return pl.pallas_call(
        body,
        out_shape=jax.ShapeDtypeStruct((M_PER, D), jnp.float32),
        in_specs=[
            pl.BlockSpec(memory_space=pltpu.VMEM),
            pl.BlockSpec(memory_space=pltpu.VMEM),
        ],
        out_specs=pl.BlockSpec(memory_space=pltpu.VMEM),
        scratch_shapes=scratch_shapes,
        compiler_params=pltpu.CompilerParams(collective_id=0),
    )(x, gamma2d)
